# Initial kernel scaffold; baseline (speedup 1.0000x reference)
import jax
import jax.numpy as jnp
from jax.experimental import pallas as pl


def kernel(input_ids, weight):
    raise NotImplementedError("write your pallas kernel here")


# speedup vs baseline: 3.5332x; 3.5332x over previous
import functools

import jax
import jax.numpy as jnp
from jax import lax
from jax.experimental import pallas as pl
from jax.experimental.pallas import tpu as pltpu
from jax.experimental.pallas import tpu_sc as plsc

BATCH = 4096
SEQ = 200
EMB_DIM = 64

NC = 2
NS = 16
NW = NC * NS

TOT = BATCH * SEQ
ROWS_PER_W = TOT // NW
G = 128
NSTEPS = ROWS_PER_W // G


def _make_gather():
    mesh = plsc.VectorSubcoreMesh(core_axis_name="c", subcore_axis_name="s")

    @functools.partial(
        pl.kernel,
        mesh=mesh,
        out_type=jax.ShapeDtypeStruct((TOT, EMB_DIM), jnp.float32),
        scratch_types=[
            pltpu.VMEM((NSTEPS, G), jnp.int32),
            pltpu.VMEM((G, EMB_DIM), jnp.float32),
        ],
        compiler_params=pltpu.CompilerParams(use_tc_tiling_on_sc=False),
    )
    def gather_kernel(table_hbm, ids_hbm, out_hbm, idx_v, rows_v):
        wid = lax.axis_index("s") * NC + lax.axis_index("c")
        base = wid * ROWS_PER_W
        pltpu.sync_copy(ids_hbm.at[wid], idx_v)

        def step(c, carry):
            pltpu.sync_copy(table_hbm.at[idx_v.at[c]], rows_v)
            pltpu.sync_copy(rows_v, out_hbm.at[pl.ds(base + c * G, G)])
            return carry

        lax.fori_loop(0, NSTEPS, step, 0)

    return gather_kernel


_gather = _make_gather()


def kernel(input_ids, weight):
    ids = input_ids.reshape(NW, NSTEPS, G).astype(jnp.int32)
    out = _gather(weight, ids)
    return out.reshape(BATCH, SEQ, EMB_DIM)


# speedup vs baseline: 4.2600x; 1.2057x over previous
import functools

import jax
import jax.numpy as jnp
from jax import lax
from jax.experimental import pallas as pl
from jax.experimental.pallas import tpu as pltpu
from jax.experimental.pallas import tpu_sc as plsc

BATCH = 4096
SEQ = 200
EMB_DIM = 64

NC = 2
NS = 16
NW = NC * NS

TOT = BATCH * SEQ
ROWS_PER_W = TOT // NW
G = 128
NSTEPS = ROWS_PER_W // G
NBUF = 8
PF = 6


def _make_gather():
    mesh = plsc.VectorSubcoreMesh(core_axis_name="c", subcore_axis_name="s")

    @functools.partial(
        pl.kernel,
        mesh=mesh,
        out_type=jax.ShapeDtypeStruct((TOT, EMB_DIM), jnp.float32),
        scratch_types=[
            pltpu.VMEM((NSTEPS, G), jnp.int32),
            pltpu.VMEM((NBUF, G, EMB_DIM), jnp.float32),
            pltpu.SemaphoreType.DMA((NBUF,)),
            pltpu.SemaphoreType.DMA((NBUF,)),
        ],
        compiler_params=pltpu.CompilerParams(use_tc_tiling_on_sc=False),
    )
    def gather_kernel(table_hbm, ids_hbm, out_hbm, idx_v, rows_v, gsem, osem):
        wid = lax.axis_index("s") * NC + lax.axis_index("c")
        base = wid * ROWS_PER_W
        pltpu.sync_copy(ids_hbm.at[wid], idx_v)

        def start_gather(g, s):
            pltpu.async_copy(table_hbm.at[idx_v.at[g]], rows_v.at[s], gsem.at[s])

        def wait_gather(s):
            pltpu.make_async_copy(
                table_hbm.at[pl.ds(0, G)], rows_v.at[s], gsem.at[s]).wait()

        def start_out(g, s):
            pltpu.async_copy(
                rows_v.at[s], out_hbm.at[pl.ds(base + g * G, G)], osem.at[s])

        def wait_out(s):
            pltpu.make_async_copy(
                rows_v.at[s], out_hbm.at[pl.ds(base, G)], osem.at[s]).wait()

        for g in range(PF):
            start_gather(g, g % NBUF)
        for g in range(2):
            wait_gather(g % NBUF)
            start_out(g, g % NBUF)
            start_gather(g + PF, (g + PF) % NBUF)

        def blk_body(blk, carry):
            for b in range(NBUF):
                g = 2 + blk * NBUF + b
                s = (2 + b) % NBUF
                wait_gather(s)
                start_out(g, s)
                s2 = (2 + b + PF) % NBUF
                wait_out(s2)
                start_gather(g + PF, s2)
            return carry

        lax.fori_loop(0, (NSTEPS - PF - 2) // NBUF, blk_body, 0)

        for g in range(NSTEPS - PF, NSTEPS):
            wait_gather(g % NBUF)
            start_out(g, g % NBUF)
        for g in range(NSTEPS - NBUF, NSTEPS):
            wait_out(g % NBUF)

    return gather_kernel


_gather = _make_gather()


def kernel(input_ids, weight):
    ids = input_ids.reshape(NW, NSTEPS, G).astype(jnp.int32)
    out = _gather(weight, ids)
    return out.reshape(BATCH, SEQ, EMB_DIM)
